# baseline (speedup 1.0000x reference)
import jax
import jax.numpy as jnp
from jax import lax
from jax.experimental import pallas as pl
from jax.experimental.pallas import tpu as pltpu
from jax.experimental.pallas import tpu_sc as plsc

_BATCH = 4096
_SEQ = 200
_EMBED = 32
_NW = 32
_BBLK = _BATCH // _NW
_NBUF = 4


_CHUNK_F = _BBLK * _EMBED


def _gather_kernel(x_hbm, table_hbm, pos_hbm, out_hbm,
                   idx_v, pos_v, r0, r1, r2, r3,
                   o0, o1, o2, o3,
                   gs0, gs1, gs2, gs3,
                   ss0, ss1, ss2, ss3):
    wid = lax.axis_index("c") * 16 + lax.axis_index("s")
    pltpu.sync_copy(x_hbm.at[wid], idx_v)
    pltpu.sync_copy(pos_hbm, pos_v)
    gbuf = (r0, r1, r2, r3)
    obuf = (o0, o1, o2, o3)
    gsem = (gs0, gs1, gs2, gs3)
    ssem = (ss0, ss1, ss2, ss3)

    def start_gather(n, rbuf, sem):
        pltpu.async_copy(table_hbm.at[idx_v.at[n]], rbuf, sem)

    def wait_gather(rbuf, sem):
        pltpu.make_async_copy(table_hbm.at[pl.ds(0, _BBLK)], rbuf, sem).wait()

    def start_store(n, ob, sem):
        pltpu.async_copy(
            ob, out_hbm.at[pl.ds((n * _NW + wid) * _CHUNK_F, _CHUNK_F)], sem)

    def wait_store(ob, sem):
        pltpu.make_async_copy(ob, out_hbm.at[pl.ds(0, _CHUNK_F)], sem).wait()

    start_gather(0, gbuf[0], gsem[0])
    start_gather(1, gbuf[1], gsem[1])

    @pl.loop(0, _SEQ // _NBUF)
    def block_group(gi):
        for j in range(_NBUF):
            n = _NBUF * gi + j
            nb = (j + 2) % _NBUF
            rbuf = gbuf[j]
            ob = obuf[j]

            @pl.when(n + 2 < _SEQ)
            def _prefetch():
                start_gather(n + 2, gbuf[nb], gsem[nb])

            wait_gather(rbuf, gsem[j])

            @pl.when(n >= _NBUF)
            def _drain_self():
                wait_store(ob, ssem[j])

            pv_lo = pos_v[n, pl.ds(0, 16)]
            pv_hi = pos_v[n, pl.ds(16, 16)]

            @plsc.parallel_loop(0, _BBLK, 1, unroll=8)
            def pos_add(r):
                ob[pl.ds(r * _EMBED, 16)] = rbuf[r, pl.ds(0, 16)] + pv_lo
                ob[pl.ds(r * _EMBED + 16, 16)] = rbuf[r, pl.ds(16, 16)] + pv_hi

            start_store(n, ob, ssem[j])

    for j in range(_NBUF):
        wait_store(obuf[j], ssem[j])


def _epi_kernel(y_ref, o_ref):
    v = y_ref[...]
    parts = [v[:, :, 32 * q:32 * (q + 1)] for q in range(4)]
    s = jnp.concatenate(parts, axis=0)
    g_i = lax.broadcasted_iota(jnp.int32, (4, 32, 128), 1)
    j_i = lax.broadcasted_iota(jnp.int32, (4, 32, 128), 2)
    q_i = lax.broadcasted_iota(jnp.int32, (4, 32, 128), 0)
    e4 = (j_i == 4 * g_i + q_i).astype(jnp.float32)
    e_all = jnp.broadcast_to(e4[:, None], (4, 32, 32, 128)).reshape(
        128, 32, 128)
    o = lax.dot_general(s, e_all, (((1,), (1,)), ((0,), (0,))),
                        precision=lax.Precision.HIGHEST,
                        preferred_element_type=jnp.float32)
    o = o.reshape(4, 32, 32, 128).sum(axis=0)
    t = o.reshape(32, 4, 8, 128).transpose(1, 0, 2, 3)
    o_ref[...] = t[None]


@jax.jit
def kernel(x, word_table, pos_table):
    B, N = x.shape
    xq = x.reshape(_NW, _BBLK, N).transpose(0, 2, 1)
    xq = xq.astype(jnp.int32)
    mesh = plsc.VectorSubcoreMesh(core_axis_name="c", subcore_axis_name="s")
    run = pl.kernel(
        _gather_kernel,
        out_type=jax.ShapeDtypeStruct((_SEQ * _NW * _CHUNK_F,), jnp.float32),
        mesh=mesh,
        scratch_types=(
            [pltpu.VMEM((_SEQ, _BBLK), jnp.int32),
             pltpu.VMEM((_SEQ, _EMBED), jnp.float32)]
            + [pltpu.VMEM((_BBLK, _EMBED), jnp.float32) for _ in range(_NBUF)]
            + [pltpu.VMEM((_CHUNK_F,), jnp.float32) for _ in range(_NBUF)]
            + [pltpu.SemaphoreType.DMA for _ in range(2 * _NBUF)]
        ),
        compiler_params=pltpu.CompilerParams(use_tc_tiling_on_sc=False),
    )
    rows = run(xq, word_table, pos_table)
    y = rows.reshape(_SEQ * _NW, _EMBED, _BBLK)
    z = pl.pallas_call(
        _epi_kernel,
        grid=(_SEQ,),
        in_specs=[pl.BlockSpec((_NW, _EMBED, _BBLK), lambda k: (k, 0, 0))],
        out_specs=pl.BlockSpec((1, 4, _NW, 8, _BBLK),
                               lambda k: (k, 0, 0, 0, 0)),
        out_shape=jax.ShapeDtypeStruct((_SEQ, 4, _NW, 8, _BBLK), jnp.float32),
    )(y)
    return z.transpose(2, 4, 0, 1, 3).reshape(B, N, _EMBED)


# speedup vs baseline: 1.4361x; 1.4361x over previous
import jax
import jax.numpy as jnp
from jax import lax
from jax.experimental import pallas as pl
from jax.experimental.pallas import tpu as pltpu
from jax.experimental.pallas import tpu_sc as plsc

_BATCH = 4096
_SEQ = 200
_EMBED = 32
_NW = 32
_BBLK = _BATCH // _NW
_NBUF = 4


_CHUNK_F = _BBLK * _EMBED


def _gather_kernel(x_hbm, table_hbm, pos_hbm, out_hbm,
                   idx_v, pos_v, r0, r1, r2, r3,
                   o0, o1, o2, o3,
                   gs0, gs1, gs2, gs3,
                   ss0, ss1, ss2, ss3):
    wid = lax.axis_index("c") * 16 + lax.axis_index("s")
    pltpu.sync_copy(x_hbm.at[wid], idx_v)
    pltpu.sync_copy(pos_hbm, pos_v)
    gbuf = (r0, r1, r2, r3)
    obuf = (o0, o1, o2, o3)
    gsem = (gs0, gs1, gs2, gs3)
    ssem = (ss0, ss1, ss2, ss3)

    def start_gather(n, rbuf, sem):
        pltpu.async_copy(table_hbm.at[idx_v.at[n]], rbuf, sem)

    def wait_gather(rbuf, sem):
        pltpu.make_async_copy(table_hbm.at[pl.ds(0, _BBLK)], rbuf, sem).wait()

    def start_store(n, ob, sem):
        pltpu.async_copy(
            ob, out_hbm.at[pl.ds((n * _NW + wid) * _CHUNK_F, _CHUNK_F)], sem)

    def wait_store(ob, sem):
        pltpu.make_async_copy(ob, out_hbm.at[pl.ds(0, _CHUNK_F)], sem).wait()

    start_gather(0, gbuf[0], gsem[0])
    start_gather(1, gbuf[1], gsem[1])

    @pl.loop(0, _SEQ // _NBUF)
    def block_group(gi):
        for j in range(_NBUF):
            n = _NBUF * gi + j
            nb = (j + 2) % _NBUF
            rbuf = gbuf[j]
            ob = obuf[j]

            @pl.when(n + 2 < _SEQ)
            def _prefetch():
                start_gather(n + 2, gbuf[nb], gsem[nb])

            wait_gather(rbuf, gsem[j])

            @pl.when(n >= _NBUF)
            def _drain_self():
                wait_store(ob, ssem[j])

            pv_lo = pos_v[n, pl.ds(0, 16)]
            pv_hi = pos_v[n, pl.ds(16, 16)]

            @plsc.parallel_loop(0, _BBLK, 1, unroll=8)
            def pos_add(r):
                ob[pl.ds(r * _EMBED, 16)] = rbuf[r, pl.ds(0, 16)] + pv_lo
                ob[pl.ds(r * _EMBED + 16, 16)] = rbuf[r, pl.ds(16, 16)] + pv_hi

            start_store(n, ob, ssem[j])

    for j in range(_NBUF):
        wait_store(obuf[j], ssem[j])


def _epi_kernel(y_ref, o_ref):
    v = y_ref[...]
    parts = [v[:, :, 32 * q:32 * (q + 1)] for q in range(4)]
    s = jnp.concatenate(parts, axis=0)
    g_i = lax.broadcasted_iota(jnp.int32, (4, 32, 128), 1)
    j_i = lax.broadcasted_iota(jnp.int32, (4, 32, 128), 2)
    q_i = lax.broadcasted_iota(jnp.int32, (4, 32, 128), 0)
    e4 = (j_i == 4 * g_i + q_i).astype(jnp.float32)
    e_all = jnp.broadcast_to(e4[:, None], (4, 32, 32, 128)).reshape(
        128, 32, 128)
    o = lax.dot_general(s, e_all, (((1,), (1,)), ((0,), (0,))),
                        preferred_element_type=jnp.float32)
    o = o.reshape(4, 32, 32, 128).sum(axis=0)
    t = o.reshape(32, 4, 8, 128).transpose(1, 0, 2, 3)
    o_ref[...] = t[None]


@jax.jit
def kernel(x, word_table, pos_table):
    B, N = x.shape
    xq = x.reshape(_NW, _BBLK, N).transpose(0, 2, 1)
    xq = xq.astype(jnp.int32)
    mesh = plsc.VectorSubcoreMesh(core_axis_name="c", subcore_axis_name="s")
    run = pl.kernel(
        _gather_kernel,
        out_type=jax.ShapeDtypeStruct((_SEQ * _NW * _CHUNK_F,), jnp.float32),
        mesh=mesh,
        scratch_types=(
            [pltpu.VMEM((_SEQ, _BBLK), jnp.int32),
             pltpu.VMEM((_SEQ, _EMBED), jnp.float32)]
            + [pltpu.VMEM((_BBLK, _EMBED), jnp.float32) for _ in range(_NBUF)]
            + [pltpu.VMEM((_CHUNK_F,), jnp.float32) for _ in range(_NBUF)]
            + [pltpu.SemaphoreType.DMA for _ in range(2 * _NBUF)]
        ),
        compiler_params=pltpu.CompilerParams(use_tc_tiling_on_sc=False),
    )
    rows = run(xq, word_table, pos_table)
    y = rows.reshape(_SEQ * _NW, _EMBED, _BBLK)
    z = pl.pallas_call(
        _epi_kernel,
        grid=(_SEQ,),
        in_specs=[pl.BlockSpec((_NW, _EMBED, _BBLK), lambda k: (k, 0, 0))],
        out_specs=pl.BlockSpec((1, 4, _NW, 8, _BBLK),
                               lambda k: (k, 0, 0, 0, 0)),
        out_shape=jax.ShapeDtypeStruct((_SEQ, 4, _NW, 8, _BBLK), jnp.float32),
    )(y)
    return z.transpose(2, 4, 0, 1, 3).reshape(B, N, _EMBED)
